# Initial kernel scaffold; baseline (speedup 1.0000x reference)
import jax
import jax.numpy as jnp
from jax.experimental import pallas as pl


def kernel(x):
    raise NotImplementedError("write your pallas kernel here")


# speedup vs baseline: 7.2494x; 7.2494x over previous
import functools

import jax
import jax.numpy as jnp
from jax import lax
from jax.experimental import pallas as pl
from jax.experimental.pallas import tpu as pltpu
from jax.experimental.pallas import tpu_sc as plsc

B, N, F = 64, 4096, 128
L = 16
NC, NS = 2, 16
NW = NC * NS
BPW = B // NW
CHUNK = 512
NCH = N // CHUNK
NV = F // L


def _sc_segment_max(x):
    mesh = plsc.VectorSubcoreMesh(core_axis_name="c", subcore_axis_name="s")

    @functools.partial(
        pl.kernel,
        mesh=mesh,
        out_type=jax.ShapeDtypeStruct((B, F), jnp.float32),
        scratch_types=[
            pltpu.VMEM((CHUNK, F), jnp.float32),
            pltpu.VMEM((BPW, F), jnp.float32),
        ],
    )
    def k(x_hbm, out_hbm, buf, acc):
        wid = lax.axis_index("s") * NC + lax.axis_index("c")
        base = wid * BPW
        for bi in range(BPW):
            accs = tuple(jnp.full((L,), -jnp.inf, jnp.float32)
                         for _ in range(NV))
            for c in range(NCH):
                pltpu.sync_copy(
                    x_hbm.at[base + bi, pl.ds(c * CHUNK, CHUNK)], buf)

                def row_body(r, a):
                    return tuple(
                        jnp.maximum(av, buf[r, pl.ds(L * f, L)])
                        for f, av in enumerate(a)
                    )
                accs = lax.fori_loop(0, CHUNK, row_body, accs)
            for f in range(NV):
                acc[bi, pl.ds(L * f, L)] = accs[f]
        pltpu.sync_copy(acc, out_hbm.at[pl.ds(base, BPW)])

    return k(x)


def kernel(x):
    return _sc_segment_max(x)


# speedup vs baseline: 9.7699x; 1.3477x over previous
import functools

import jax
import jax.numpy as jnp
from jax import lax
from jax.experimental import pallas as pl
from jax.experimental.pallas import tpu as pltpu
from jax.experimental.pallas import tpu_sc as plsc

B, N, F = 64, 4096, 128
L = 16
NC, NS = 2, 16
NW = NC * NS
BPW = B // NW
CHUNK = 256
NCH = N // CHUNK
TOT = BPW * NCH
NV = F // L
U = 8


def _sc_segment_max(x):
    mesh = plsc.VectorSubcoreMesh(core_axis_name="c", subcore_axis_name="s")

    @functools.partial(
        pl.kernel,
        mesh=mesh,
        out_type=jax.ShapeDtypeStruct((B, F), jnp.float32),
        scratch_types=[
            pltpu.VMEM((2, CHUNK, F), jnp.float32),
            pltpu.VMEM((BPW, F), jnp.float32),
            pltpu.SemaphoreType.DMA,
            pltpu.SemaphoreType.DMA,
        ],
    )
    def k(x_hbm, out_hbm, buf, acc, sem0, sem1):
        sems = (sem0, sem1)
        wid = lax.axis_index("s") * NC + lax.axis_index("c")
        base = wid * BPW

        def start(j):
            bi, c = divmod(j, NCH)
            slot = j % 2
            return pltpu.async_copy(
                x_hbm.at[base + bi, pl.ds(c * CHUNK, CHUNK)],
                buf.at[slot], sems[slot])

        cps = {0: start(0)}
        for bi in range(BPW):
            accs = tuple(jnp.full((L,), -jnp.inf, jnp.float32)
                         for _ in range(NV))
            for c in range(NCH):
                j = bi * NCH + c
                if j + 1 < TOT:
                    cps[j + 1] = start(j + 1)
                cps.pop(j).wait()
                slot = j % 2

                def row_body(r, a, slot=slot):
                    out = []
                    for f in range(NV):
                        m = a[f]
                        for u in range(U):
                            m = jnp.maximum(
                                m, buf[slot, r * U + u, pl.ds(L * f, L)])
                        out.append(m)
                    return tuple(out)

                accs = lax.fori_loop(0, CHUNK // U, row_body, accs)
            for f in range(NV):
                acc[bi, pl.ds(L * f, L)] = accs[f]
        pltpu.sync_copy(acc, out_hbm.at[pl.ds(base, BPW)])

    return k(x)


def kernel(x):
    return _sc_segment_max(x)


# speedup vs baseline: 10.4131x; 1.0658x over previous
import functools

import jax
import jax.numpy as jnp
from jax import lax
from jax.experimental import pallas as pl
from jax.experimental.pallas import tpu as pltpu
from jax.experimental.pallas import tpu_sc as plsc

B, N, F = 64, 4096, 128
L = 16
NC, NS = 2, 16
NW = NC * NS
BPW = B // NW
CHUNK = 256
NCH = N // CHUNK
TOT = BPW * NCH
NV = F // L
U = 4


def _sc_segment_max(x):
    mesh = plsc.VectorSubcoreMesh(core_axis_name="c", subcore_axis_name="s")

    @functools.partial(
        pl.kernel,
        mesh=mesh,
        out_type=jax.ShapeDtypeStruct((B, F), jnp.float32),
        scratch_types=[
            pltpu.VMEM((2, CHUNK, F), jnp.float32),
            pltpu.VMEM((BPW, F), jnp.float32),
            pltpu.SemaphoreType.DMA,
            pltpu.SemaphoreType.DMA,
        ],
    )
    def k(x_hbm, out_hbm, buf, acc, sem0, sem1):
        sems = (sem0, sem1)
        wid = lax.axis_index("s") * NC + lax.axis_index("c")
        base = wid * BPW

        def start(j):
            bi, c = divmod(j, NCH)
            slot = j % 2
            return pltpu.async_copy(
                x_hbm.at[base + bi, pl.ds(c * CHUNK, CHUNK)],
                buf.at[slot], sems[slot])

        cps = {0: start(0)}
        for bi in range(BPW):
            accs = tuple(jnp.full((L,), -jnp.inf, jnp.float32)
                         for _ in range(NV))
            for c in range(NCH):
                j = bi * NCH + c
                if j + 1 < TOT:
                    cps[j + 1] = start(j + 1)
                cps.pop(j).wait()
                slot = j % 2

                def row_body(r, a, slot=slot):
                    out = []
                    for f in range(NV):
                        m = a[f]
                        for u in range(U):
                            m = jnp.maximum(
                                m, buf[slot, r * U + u, pl.ds(L * f, L)])
                        out.append(m)
                    return tuple(out)

                accs = lax.fori_loop(0, CHUNK // U, row_body, accs)
            for f in range(NV):
                acc[bi, pl.ds(L * f, L)] = accs[f]
        pltpu.sync_copy(acc, out_hbm.at[pl.ds(base, BPW)])

    return k(x)


def kernel(x):
    return _sc_segment_max(x)
